# baseline (speedup 1.0000x reference)
import functools

import jax
import jax.numpy as jnp
from jax import lax
from jax.experimental import pallas as pl
from jax.experimental.pallas import tpu as pltpu
from jax.experimental.pallas import tpu_sc as plsc

N = 2226
E = 6525
R = 1800

NC = 2
NS = 16
NW = NC * NS

PER_R = 64
PER_E = 208
PER_N = 80

_F32 = jnp.float32
_I32 = jnp.int32


def _sqrt16(ss):
    i = lax.bitcast_convert_type(ss, _I32)
    y = lax.bitcast_convert_type(
        jnp.int32(0x5F3759DF) - lax.shift_right_logical(i, 1), _F32)
    for _ in range(3):
        y = y * (1.5 - 0.5 * ss * y * y)
    return jnp.where(ss > 0.0, ss * y, 0.0)


def _body(pos_h, str_h, consts_h, dir_h, lene_h, act_h,
          rope_h, refl_h, edge_h,
          loss_o, c_o, ceq_o, stre_o,
          pos_v, act_v, dir_v, str_v, rope_v, refl_v, edge_v, lene_v,
          consts_v,
          loss_s, c_s, ceq_s, stre_s, sem):
    wid = lax.axis_index("s") * NC + lax.axis_index("c")

    pairs = [(pos_h, pos_v), (act_h, act_v), (dir_h, dir_v),
             (str_h, str_v), (rope_h, rope_v), (refl_h, refl_v),
             (edge_h, edge_v), (lene_h, lene_v), (consts_h, consts_v)]
    handles = [pltpu.async_copy(src, dst, sem) for src, dst in pairs]
    for h in handles:
        h.wait()

    iota = lax.iota(_I32, 16)

    cv = consts_v[...]

    def _spl(k):
        return jnp.sum(jnp.where(iota == k, cv, 0.0))

    r00, r01, r02 = _spl(0), _spl(1), _spl(2)
    r10, r11, r12 = _spl(3), _spl(4), _spl(5)
    r20, r21, r22 = _spl(6), _spl(7), _spl(8)
    fx, fy, fz = _spl(9), _spl(10), _spl(11)
    bias2 = _spl(12) * 2.0 + 440.0

    base_r = wid * PER_R
    for j in range(PER_R // 16):
        ii = jnp.minimum(base_r + j * 16 + iota, R - 1)
        ridx = plsc.load_gather(refl_v, [ii]) * 3
        px = plsc.load_gather(pos_v, [ridx])
        py = plsc.load_gather(pos_v, [ridx + 1])
        pz = plsc.load_gather(pos_v, [ridx + 2])
        rx = px * r00 + py * r10 + pz * r20
        ry = px * r01 + py * r11 + pz * r21
        rz = px * r02 + py * r12 + pz * r22
        ex = rx - fx
        ey = ry - fy
        ez = rz - fz
        dis = _sqrt16(ex * ex + ey * ey + ez * ez)
        t = jnp.abs(dis - (rz + bias2)) - 1.0
        loss_s[pl.ds(j * 16, 16)] = jnp.maximum(t, 0.0)

    base_e = wid * PER_E
    for j in range(PER_E // 16):
        ii = jnp.minimum(base_e + j * 16 + iota, E - 1)
        ia = plsc.load_gather(edge_v, [ii * 2]) * 3
        ib = plsc.load_gather(edge_v, [ii * 2 + 1]) * 3
        dx = plsc.load_gather(pos_v, [ia]) - plsc.load_gather(pos_v, [ib])
        dy = (plsc.load_gather(pos_v, [ia + 1])
              - plsc.load_gather(pos_v, [ib + 1]))
        dz = (plsc.load_gather(pos_v, [ia + 2])
              - plsc.load_gather(pos_v, [ib + 2]))
        lens = _sqrt16(dx * dx + dy * dy + dz * dz)
        le = plsc.load_gather(lene_v, [ii])
        c = jnp.maximum(jnp.abs(lens - le) - 0.007 * le, 0.0) * 100.0
        c_s[pl.ds(j * 16, 16)] = c

    base_n = wid * PER_N
    for j in range(PER_N // 16):
        ii = jnp.minimum(base_n + j * 16 + iota, N - 1)
        i3 = ii * 3
        s = plsc.load_gather(str_v, [ii])
        rx = (plsc.load_gather(act_v, [i3])
              + plsc.load_gather(dir_v, [i3]) * s
              - plsc.load_gather(pos_v, [i3]))
        ry = (plsc.load_gather(act_v, [i3 + 1])
              + plsc.load_gather(dir_v, [i3 + 1]) * s
              - plsc.load_gather(pos_v, [i3 + 1]))
        rz = (plsc.load_gather(act_v, [i3 + 2])
              + plsc.load_gather(dir_v, [i3 + 2]) * s
              - plsc.load_gather(pos_v, [i3 + 2]))
        nn = _sqrt16(rx * rx + ry * ry + rz * rz)
        lr = plsc.load_gather(rope_v, [ii])
        ceq_s[pl.ds(j * 16, 16)] = jnp.abs(lr - nn) * 100.0
        stre_s[pl.ds(j * 16, 16)] = jnp.maximum(jnp.abs(s) - 0.6, 0.0)

    pltpu.sync_copy(loss_s, loss_o.at[pl.ds(base_r, PER_R)])
    pltpu.sync_copy(c_s, c_o.at[pl.ds(base_e, PER_E)])
    pltpu.sync_copy(ceq_s, ceq_o.at[pl.ds(base_n, PER_N)])
    pltpu.sync_copy(stre_s, stre_o.at[pl.ds(base_n, PER_N)])


_sc_call = functools.partial(
    pl.kernel,
    out_type=[
        jax.ShapeDtypeStruct((NW * PER_R,), _F32),
        jax.ShapeDtypeStruct((NW * PER_E,), _F32),
        jax.ShapeDtypeStruct((NW * PER_N,), _F32),
        jax.ShapeDtypeStruct((NW * PER_N,), _F32),
    ],
    mesh=plsc.VectorSubcoreMesh(core_axis_name="c", subcore_axis_name="s",
                                num_cores=NC, num_subcores=NS),
    compiler_params=pltpu.CompilerParams(needs_layout_passes=False),
    scratch_types=[
        pltpu.VMEM((N * 3,), _F32),
        pltpu.VMEM((N * 3,), _F32),
        pltpu.VMEM((N * 3,), _F32),
        pltpu.VMEM((N,), _F32),
        pltpu.VMEM((N,), _F32),
        pltpu.VMEM((R,), _I32),
        pltpu.VMEM((E * 2,), _I32),
        pltpu.VMEM((E,), _F32),
        pltpu.VMEM((16,), _F32),
        pltpu.VMEM((PER_R,), _F32),
        pltpu.VMEM((PER_E,), _F32),
        pltpu.VMEM((PER_N,), _F32),
        pltpu.VMEM((PER_N,), _F32),
        pltpu.SemaphoreType.DMA,
    ],
)(_body)


def kernel(pos, stretch, bias, rotm, direction, focus, len_edges, act_up,
           len_rope, refl_idx, all_edges):
    consts = jnp.concatenate([rotm.reshape(9), focus.reshape(3),
                              bias.reshape(1), jnp.zeros((3,), _F32)])
    loss_p, c_p, ceq_p, stre_p = _sc_call(
        pos.reshape(-1), stretch.reshape(-1), consts,
        direction.reshape(-1), len_edges, act_up.reshape(-1),
        len_rope, refl_idx.astype(_I32),
        all_edges.astype(_I32).reshape(-1))
    return jnp.concatenate([loss_p[:R], c_p[:E], ceq_p[:N], stre_p[:N]])
